# baseline (speedup 1.0000x reference)
import functools

import jax
import jax.numpy as jnp
import numpy as np
from jax.experimental import pallas as pl
from jax.experimental.pallas import tpu as pltpu


@functools.lru_cache(maxsize=None)
def _conn_mask(board):
    N = int(np.prod(board))
    coords = np.stack(np.unravel_index(np.arange(N), board), axis=-1)
    delta = np.abs(coords[None, :, :] - coords[:, None, :])
    mx = delta.max(axis=-1)
    connected = (mx > 0) & np.all((delta == 0) | (delta == mx[..., None]), axis=-1)
    return connected.astype(np.int8)


def _attn_kernel(xq_ref, xk_ref, xv_ref, wq_ref, bq_ref, wk_ref, bk_ref,
                 wv_ref, bv_ref, mask_hbm, out_ref, vv_sc, mask_sc, sem,
                 *, out_dim, scale):
    b = pl.program_id(0)

    @pl.when(b == 0)
    def _start_mask_copy():
        pltpu.make_async_copy(mask_hbm, mask_sc, sem).start()

    q = ((jnp.dot(xq_ref[0], wq_ref[...],
                  preferred_element_type=jnp.float32)
          + bq_ref[...]) * scale).astype(jnp.bfloat16)
    k = (jnp.dot(xk_ref[0], wk_ref[...],
                 preferred_element_type=jnp.float32)
         + bk_ref[...]).astype(jnp.bfloat16)
    v = (jnp.dot(xv_ref[0], wv_ref[...],
                 preferred_element_type=jnp.float32)
         + bv_ref[...]).astype(jnp.bfloat16)
    vv_sc[:, :out_dim] = v
    vv_sc[:, out_dim:] = jnp.ones_like(vv_sc[:, out_dim:])
    s = jax.lax.dot_general(q, k, (((1,), (1,)), ((), ())),
                            preferred_element_type=jnp.float32)

    @pl.when(b == 0)
    def _wait_mask_copy():
        pltpu.make_async_copy(mask_hbm, mask_sc, sem).wait()

    p = jnp.where(mask_sc[...] != 0, jnp.exp2(s), 0.0).astype(jnp.bfloat16)
    o = jnp.dot(p, vv_sc[...], preferred_element_type=jnp.float32)
    out_ref[0] = o[:, :out_dim] / o[:, out_dim:out_dim + 1]


def kernel(query_X, key_X, value_X, Wq, bq, Wk, bk, Wv, bv):
    B = query_X.shape[0]
    board = tuple(int(d) for d in query_X.shape[1:-1])
    in_dim = query_X.shape[-1]
    cmp_dim = Wq.shape[1]
    out_dim = Wv.shape[1]
    N = int(np.prod(board))
    scale = float(np.log2(np.e)) / (cmp_dim ** 0.5)

    mask = jnp.asarray(_conn_mask(board))
    xq = query_X.reshape(B, N, in_dim)
    xk = key_X.reshape(B, N, in_dim)
    xv = value_X.reshape(B, N, in_dim)

    batch_spec = pl.BlockSpec((1, N, in_dim), lambda b: (b, 0, 0))
    full = lambda shape: pl.BlockSpec(shape, lambda b: (0,) * len(shape))

    out = pl.pallas_call(
        functools.partial(_attn_kernel, out_dim=out_dim, scale=scale),
        grid=(B,),
        in_specs=[
            batch_spec, batch_spec, batch_spec,
            full((in_dim, cmp_dim)), full((1, cmp_dim)),
            full((in_dim, cmp_dim)), full((1, cmp_dim)),
            full((in_dim, out_dim)), full((1, out_dim)),
            pl.BlockSpec(memory_space=pl.ANY),
        ],
        out_specs=pl.BlockSpec((1, N, out_dim), lambda b: (b, 0, 0)),
        out_shape=jax.ShapeDtypeStruct((B, N, out_dim), jnp.float32),
        scratch_shapes=[pltpu.VMEM((N, out_dim + 8), jnp.bfloat16),
                        pltpu.VMEM((N, N), jnp.int8),
                        pltpu.SemaphoreType.DMA],
    )(xq, xk, xv, Wq, bq.reshape(1, cmp_dim), Wk, bk.reshape(1, cmp_dim),
      Wv, bv.reshape(1, out_dim), mask)

    return out.reshape((B,) + board + (out_dim,))


# speedup vs baseline: 1.1248x; 1.1248x over previous
import functools

import jax
import jax.numpy as jnp
import numpy as np
from jax.experimental import pallas as pl
from jax.experimental.pallas import tpu as pltpu


@functools.lru_cache(maxsize=None)
def _conn_mask(board):
    N = int(np.prod(board))
    coords = np.stack(np.unravel_index(np.arange(N), board), axis=-1)
    delta = np.abs(coords[None, :, :] - coords[:, None, :])
    mx = delta.max(axis=-1)
    connected = (mx > 0) & np.all((delta == 0) | (delta == mx[..., None]), axis=-1)
    return connected.astype(jnp.bfloat16.dtype)


def _attn_kernel(xq_ref, xk_ref, xv_ref, wq_ref, bq_ref, wk_ref, bk_ref,
                 wv_ref, bv_ref, mask_ref, out_ref, vv_sc, *, out_dim, scale):
    q = ((jnp.dot(xq_ref[0], wq_ref[...],
                  preferred_element_type=jnp.float32)
          + bq_ref[...]) * scale).astype(jnp.bfloat16)
    k = (jnp.dot(xk_ref[0], wk_ref[...],
                 preferred_element_type=jnp.float32)
         + bk_ref[...]).astype(jnp.bfloat16)
    v = (jnp.dot(xv_ref[0], wv_ref[...],
                 preferred_element_type=jnp.float32)
         + bv_ref[...]).astype(jnp.bfloat16)
    vv_sc[:, :out_dim] = v
    vv_sc[:, out_dim:] = jnp.ones_like(vv_sc[:, out_dim:])
    s = jax.lax.dot_general(q, k, (((1,), (1,)), ((), ())),
                            preferred_element_type=jnp.float32)
    p = jnp.exp2(s).astype(jnp.bfloat16) * mask_ref[...]
    o = jnp.dot(p, vv_sc[...], preferred_element_type=jnp.float32)
    out_ref[0] = o[:, :out_dim] / o[:, out_dim:out_dim + 1]


def kernel(query_X, key_X, value_X, Wq, bq, Wk, bk, Wv, bv):
    B = query_X.shape[0]
    board = tuple(int(d) for d in query_X.shape[1:-1])
    in_dim = query_X.shape[-1]
    cmp_dim = Wq.shape[1]
    out_dim = Wv.shape[1]
    N = int(np.prod(board))
    scale = float(np.log2(np.e)) / (cmp_dim ** 0.5)

    mask = jnp.asarray(_conn_mask(board))
    xq = query_X.reshape(B, N, in_dim)
    xk = key_X.reshape(B, N, in_dim)
    xv = value_X.reshape(B, N, in_dim)

    batch_spec = pl.BlockSpec((1, N, in_dim), lambda b: (b, 0, 0))
    full = lambda shape: pl.BlockSpec(shape, lambda b: (0,) * len(shape))

    out = pl.pallas_call(
        functools.partial(_attn_kernel, out_dim=out_dim, scale=scale),
        grid=(B,),
        in_specs=[
            batch_spec, batch_spec, batch_spec,
            full((in_dim, cmp_dim)), full((1, cmp_dim)),
            full((in_dim, cmp_dim)), full((1, cmp_dim)),
            full((in_dim, out_dim)), full((1, out_dim)),
            full((N, N)),
        ],
        out_specs=pl.BlockSpec((1, N, out_dim), lambda b: (b, 0, 0)),
        out_shape=jax.ShapeDtypeStruct((B, N, out_dim), jnp.float32),
        scratch_shapes=[pltpu.VMEM((N, out_dim + 8), jnp.bfloat16)],
    )(xq, xk, xv, Wq, bq.reshape(1, cmp_dim), Wk, bk.reshape(1, cmp_dim),
      Wv, bv.reshape(1, out_dim), mask)

    return out.reshape((B,) + board + (out_dim,))


# speedup vs baseline: 1.1531x; 1.0252x over previous
import functools

import jax
import jax.numpy as jnp
import numpy as np
from jax.experimental import pallas as pl
from jax.experimental.pallas import tpu as pltpu


@functools.lru_cache(maxsize=None)
def _conn_mask(board):
    N = int(np.prod(board))
    coords = np.stack(np.unravel_index(np.arange(N), board), axis=-1)
    delta = np.abs(coords[None, :, :] - coords[:, None, :])
    mx = delta.max(axis=-1)
    connected = (mx > 0) & np.all((delta == 0) | (delta == mx[..., None]), axis=-1)
    return connected.astype(np.int8)


def _attn_kernel(xq_ref, xk_ref, xv_ref, wq_ref, bq_ref, wk_ref, bk_ref,
                 wv_ref, bv_ref, mask_ref, out_ref, vv_sc, *, out_dim, scale):
    q = ((jnp.dot(xq_ref[0], wq_ref[...],
                  preferred_element_type=jnp.float32)
          + bq_ref[...]) * scale).astype(jnp.bfloat16)
    k = (jnp.dot(xk_ref[0], wk_ref[...],
                 preferred_element_type=jnp.float32)
         + bk_ref[...]).astype(jnp.bfloat16)
    v = (jnp.dot(xv_ref[0], wv_ref[...],
                 preferred_element_type=jnp.float32)
         + bv_ref[...]).astype(jnp.bfloat16)
    vv_sc[:, :out_dim] = v
    vv_sc[:, out_dim:] = jnp.ones_like(vv_sc[:, out_dim:])
    s = jax.lax.dot_general(q, k, (((1,), (1,)), ((), ())),
                            preferred_element_type=jnp.float32)
    p = jnp.where(mask_ref[...] != 0, jnp.exp2(s), 0.0).astype(jnp.bfloat16)
    o = jnp.dot(p, vv_sc[...], preferred_element_type=jnp.float32)
    out_ref[0] = o[:, :out_dim] / o[:, out_dim:out_dim + 1]


def kernel(query_X, key_X, value_X, Wq, bq, Wk, bk, Wv, bv):
    B = query_X.shape[0]
    board = tuple(int(d) for d in query_X.shape[1:-1])
    in_dim = query_X.shape[-1]
    cmp_dim = Wq.shape[1]
    out_dim = Wv.shape[1]
    N = int(np.prod(board))
    scale = float(np.log2(np.e)) / (cmp_dim ** 0.5)

    mask = jnp.asarray(_conn_mask(board))
    xq = query_X.reshape(B, N, in_dim)
    xk = key_X.reshape(B, N, in_dim)
    xv = value_X.reshape(B, N, in_dim)

    batch_spec = pl.BlockSpec((1, N, in_dim), lambda b: (b, 0, 0))
    full = lambda shape: pl.BlockSpec(shape, lambda b: (0,) * len(shape))

    out = pl.pallas_call(
        functools.partial(_attn_kernel, out_dim=out_dim, scale=scale),
        grid=(B,),
        in_specs=[
            batch_spec, batch_spec, batch_spec,
            full((in_dim, cmp_dim)), full((1, cmp_dim)),
            full((in_dim, cmp_dim)), full((1, cmp_dim)),
            full((in_dim, out_dim)), full((1, out_dim)),
            full((N, N)),
        ],
        out_specs=pl.BlockSpec((1, N, out_dim), lambda b: (b, 0, 0)),
        out_shape=jax.ShapeDtypeStruct((B, N, out_dim), jnp.float32),
        scratch_shapes=[pltpu.VMEM((N, out_dim + 8), jnp.bfloat16)],
    )(xq, xk, xv, Wq, bq.reshape(1, cmp_dim), Wk, bk.reshape(1, cmp_dim),
      Wv, bv.reshape(1, out_dim), mask)

    return out.reshape((B,) + board + (out_dim,))
